# Initial kernel scaffold; baseline (speedup 1.0000x reference)
import jax
import jax.numpy as jnp
from jax.experimental import pallas as pl


def kernel(input_ids, attention_mask, emb_weight, fc_w, fc_b):
    raise NotImplementedError("write your pallas kernel here")


# speedup vs baseline: 1.4028x; 1.4028x over previous
import functools

import jax
import jax.numpy as jnp
from jax import lax
from jax.experimental import pallas as pl
from jax.experimental.pallas import tpu as pltpu
from jax.experimental.pallas import tpu_sc as plsc

V, D, L = 100000, 64, 2
B, S = 4096, 200

NC, NS, LANES = 2, 16, 16
NW = NC * NS
RPW = B // NW
TPW = RPW * S
DEAD = V
VP = V + 8
VBLK = 11112
NBUF = 4
G1, G2 = 104, 96


def _proj_body(emb_ref, w_ref, out_ref):
    i = pl.program_id(0)
    y = jnp.dot(emb_ref[...], w_ref[...], preferred_element_type=jnp.float32)
    rows = i * VBLK + lax.broadcasted_iota(jnp.int32, (VBLK, 16), 0)
    cols = lax.broadcasted_iota(jnp.int32, (VBLK, 16), 1)
    y = y + jnp.where(cols == 2, 1.0, 0.0)
    out_ref[...] = jnp.where(rows < V, y, 0.0)


_proj_table = pl.pallas_call(
    _proj_body,
    grid=(VP // VBLK,),
    in_specs=[
        pl.BlockSpec((VBLK, D), lambda i: (i, 0)),
        pl.BlockSpec((D, 16), lambda i: (0, 0)),
    ],
    out_specs=pl.BlockSpec((VBLK, 16), lambda i: (i, 0)),
    out_shape=jax.ShapeDtypeStruct((VP, 16), jnp.float32),
)


def _sc_body(proj_hbm, ids_hbm, msk_hbm, bias_hbm, out_hbm,
             ids_v, msk_v, rows_v, out_v, bias_v, sems):
    wid = lax.axis_index("s") * NC + lax.axis_index("c")
    row0 = wid * RPW
    tok0 = row0 * S

    pltpu.sync_copy(ids_hbm.at[pl.ds(tok0, TPW)], ids_v)
    pltpu.sync_copy(msk_hbm.at[pl.ds(tok0, TPW)], msk_v)
    pltpu.sync_copy(bias_hbm, bias_v)

    def sel_body(i, carry):
        sl = pl.ds(i * LANES, LANES)
        ids_v[sl] = jnp.where(msk_v[sl] > 0, ids_v[sl], DEAD)
        return carry
    lax.fori_loop(0, TPW // LANES, sel_body, 0)

    def issue(r, slot):
        t0 = r * S
        pltpu.make_async_copy(
            proj_hbm.at[ids_v.at[pl.ds(t0, G1)]],
            rows_v.at[slot, pl.ds(0, G1)], sems[slot]).start()
        pltpu.make_async_copy(
            proj_hbm.at[ids_v.at[pl.ds(t0 + G1, G2)]],
            rows_v.at[slot, pl.ds(G1, G2)], sems[slot]).start()

    def drain(slot):
        pltpu.make_async_copy(
            proj_hbm.at[ids_v.at[pl.ds(0, G1)]],
            rows_v.at[slot, pl.ds(0, G1)], sems[slot]).wait()
        pltpu.make_async_copy(
            proj_hbm.at[ids_v.at[pl.ds(0, G2)]],
            rows_v.at[slot, pl.ds(G1, G2)], sems[slot]).wait()

    for slot in range(NBUF):
        issue(slot, slot)

    lane2 = jnp.full((LANES,), 2, jnp.int32)
    zero = jnp.zeros((LANES,), jnp.float32)

    def group_body(g, carry):
        for slot in range(NBUF):
            r = g * NBUF + slot
            drain(slot)

            def acc_body(j, accs):
                a0, a1, a2, a3 = accs
                j4 = j * 4
                return (a0 + rows_v[slot, j4],
                        a1 + rows_v[slot, j4 + 1],
                        a2 + rows_v[slot, j4 + 2],
                        a3 + rows_v[slot, j4 + 3])
            a0, a1, a2, a3 = lax.fori_loop(
                0, S // 4, acc_body, (zero, zero, zero, zero))
            acc = (a0 + a1) + (a2 + a3)

            @pl.when(r + NBUF < RPW)
            def _():
                issue(r + NBUF, slot)

            cntv = lax.gather(
                acc, lane2[:, None],
                lax.GatherDimensionNumbers(
                    offset_dims=(), collapsed_slice_dims=(0,),
                    start_index_map=(0,)),
                (1,), mode=lax.GatherScatterMode.PROMISE_IN_BOUNDS)
            den = jnp.maximum(cntv, 1.0)
            out_v[r] = acc / den + bias_v[...]
        return carry
    lax.fori_loop(0, RPW // NBUF, group_body, 0)

    pltpu.sync_copy(out_v, out_hbm.at[pl.ds(row0, RPW)])


_sc_head = functools.partial(
    pl.kernel,
    out_type=jax.ShapeDtypeStruct((B, 16), jnp.float32),
    mesh=plsc.VectorSubcoreMesh(core_axis_name="c", subcore_axis_name="s"),
    compiler_params=pltpu.CompilerParams(use_tc_tiling_on_sc=False),
    scratch_types=[
        pltpu.VMEM((TPW,), jnp.int32),
        pltpu.VMEM((TPW,), jnp.int32),
        pltpu.VMEM((NBUF, S, 16), jnp.float32),
        pltpu.VMEM((RPW, 16), jnp.float32),
        pltpu.VMEM((16,), jnp.float32),
        [pltpu.SemaphoreType.DMA] * NBUF,
    ],
)(_sc_body)


def kernel(input_ids, attention_mask, emb_weight, fc_w, fc_b):
    w16 = jnp.zeros((D, 16), jnp.float32).at[:, :2].set(fc_w.T)
    proj = _proj_table(emb_weight, w16)
    ids_flat = input_ids.reshape(-1).astype(jnp.int32)
    msk_flat = attention_mask.reshape(-1).astype(jnp.int32)
    bias16 = jnp.zeros((16,), jnp.float32).at[:2].set(fc_b)
    out16 = _sc_head(proj, ids_flat, msk_flat, bias16)
    return out16[:, :2]


# speedup vs baseline: 21.7544x; 15.5076x over previous
import functools

import jax
import jax.numpy as jnp
from jax import lax
from jax.experimental import pallas as pl
from jax.experimental.pallas import tpu as pltpu
from jax.experimental.pallas import tpu_sc as plsc

V, D, L = 100000, 64, 2
B, S = 4096, 200

NC, NS, LANES = 2, 16, 16
DEAD = V
VP = 100096
CBLK = 5888
SP = 208
RPT = B // NS
SLAB = LANES * SP
NSLAB = RPT // LANES


def _proj_body(w_ref, emb_ref, out_ref):
    i = pl.program_id(0)
    y = lax.dot_general(w_ref[...], emb_ref[...],
                        (((1,), (1,)), ((), ())),
                        preferred_element_type=jnp.float32)
    cols = i * CBLK + lax.broadcasted_iota(jnp.int32, (L, CBLK), 1)
    out_ref[...] = jnp.where(cols < V, y, 0.0)


_proj_table = pl.pallas_call(
    _proj_body,
    grid=(VP // CBLK,),
    in_specs=[
        pl.BlockSpec((L, D), lambda i: (0, 0)),
        pl.BlockSpec((CBLK, D), lambda i: (i, 0)),
    ],
    out_specs=pl.BlockSpec((L, CBLK), lambda i: (0, i)),
    out_shape=jax.ShapeDtypeStruct((L, VP), jnp.float32),
)


def _rot(x, idx):
    return lax.gather(
        x, idx[:, None],
        lax.GatherDimensionNumbers(
            offset_dims=(), collapsed_slice_dims=(0,), start_index_map=(0,)),
        (1,), mode=lax.GatherScatterMode.PROMISE_IN_BOUNDS)


def _sc_body(tbl_hbm, ids_hbm, msk_hbm, out_hbm, tbl_v, ids_v, msk_v, out_v,
             sems):
    lgt = lax.axis_index("c")
    sid = lax.axis_index("s")
    row0 = sid * RPT
    tok0 = row0 * SP

    pltpu.sync_copy(tbl_hbm.at[lgt], tbl_v)

    def issue(slab, buf):
        off = tok0 + slab * SLAB
        pltpu.make_async_copy(
            ids_hbm.at[pl.ds(off, SLAB)], ids_v.at[buf], sems[buf]).start()
        pltpu.make_async_copy(
            msk_hbm.at[pl.ds(off, SLAB)], msk_v.at[buf], sems[buf]).start()

    def drain(buf):
        pltpu.make_async_copy(
            ids_hbm.at[pl.ds(tok0, SLAB)], ids_v.at[buf], sems[buf]).wait()
        pltpu.make_async_copy(
            msk_hbm.at[pl.ds(tok0, SLAB)], msk_v.at[buf], sems[buf]).wait()

    issue(0, 0)
    issue(1, 1)

    lane = lax.iota(jnp.int32, LANES)
    rot8 = (lane + 8) & 15
    rot4 = (lane + 4) & 15
    rot2 = (lane + 2) & 15
    rot1 = (lane + 1) & 15
    zero = jnp.zeros((LANES,), jnp.float32)

    def slab_group(g, carry):
        for buf in range(2):
            slab = g * 2 + buf
            drain(buf)

            def row_body(r, slabreg):
                rb = r * SP
                acc = zero
                for c in range(SP // LANES):
                    sl = pl.ds(rb + c * LANES, LANES)
                    sel = jnp.where(msk_v[buf, sl] > 0, ids_v[buf, sl], DEAD)
                    acc = acc + plsc.load_gather(tbl_v, [sel])
                acc = acc + _rot(acc, rot8)
                acc = acc + _rot(acc, rot4)
                acc = acc + _rot(acc, rot2)
                acc = acc + _rot(acc, rot1)
                return jnp.where(lane == r, acc, slabreg)
            slabreg = lax.fori_loop(0, LANES, row_body, zero)

            @pl.when(slab + 2 < NSLAB)
            def _():
                issue(slab + 2, buf)

            out_v[pl.ds(slab * LANES, LANES)] = slabreg
        return carry
    lax.fori_loop(0, NSLAB // 2, slab_group, 0)

    pltpu.sync_copy(out_v, out_hbm.at[lgt, pl.ds(row0, RPT)])


_sc_sums = functools.partial(
    pl.kernel,
    out_type=jax.ShapeDtypeStruct((L, B), jnp.float32),
    mesh=plsc.VectorSubcoreMesh(core_axis_name="c", subcore_axis_name="s"),
    compiler_params=pltpu.CompilerParams(
        use_tc_tiling_on_sc=False, needs_layout_passes=False),
    scratch_types=[
        pltpu.VMEM((VP,), jnp.float32),
        pltpu.VMEM((2, SLAB), jnp.int32),
        pltpu.VMEM((2, SLAB), jnp.int32),
        pltpu.VMEM((RPT,), jnp.float32),
        [pltpu.SemaphoreType.DMA] * 2,
    ],
)(_sc_body)


def _fin_body(sums_ref, msk_ref, b_ref, out_ref):
    m = msk_ref[...].astype(jnp.float32)
    inv = 1.0 / jnp.maximum(jnp.sum(m, axis=1), 1.0)
    s = sums_ref[...]
    o0 = s[0, :] * inv + b_ref[0, 0]
    o1 = s[1, :] * inv + b_ref[0, 1]
    out_ref[...] = jnp.concatenate([o0[:, None], o1[:, None]], axis=1)


_finalize = pl.pallas_call(
    _fin_body,
    in_specs=[
        pl.BlockSpec((L, B), lambda: (0, 0)),
        pl.BlockSpec((B, S), lambda: (0, 0)),
        pl.BlockSpec((1, L), lambda: (0, 0)),
    ],
    out_specs=pl.BlockSpec((B, L), lambda: (0, 0)),
    out_shape=jax.ShapeDtypeStruct((B, L), jnp.float32),
)


def kernel(input_ids, attention_mask, emb_weight, fc_w, fc_b):
    w2 = fc_w.astype(jnp.float32)
    tbl = _proj_table(w2, emb_weight)
    ids_p = jnp.pad(input_ids.astype(jnp.int32), ((0, 0), (0, SP - S)))
    msk_p = jnp.pad(attention_mask.astype(jnp.int32), ((0, 0), (0, SP - S)))
    sums = _sc_sums(tbl, ids_p.reshape(-1), msk_p.reshape(-1))
    return _finalize(sums, attention_mask.astype(jnp.int32),
                     fc_b.reshape(1, L).astype(jnp.float32))


# speedup vs baseline: 47.3560x; 2.1769x over previous
import functools

import jax
import jax.numpy as jnp
from jax import lax
from jax.experimental import pallas as pl
from jax.experimental.pallas import tpu as pltpu
from jax.experimental.pallas import tpu_sc as plsc

V, D, L = 100000, 64, 2
B, S = 4096, 200

NC, NS, LANES = 2, 16, 16
NW = NC * NS
DEAD = V
VP = 102400
CBLK = 51200
SP = 208
RPT = B // NW
PBLK = 512
NSLAB = RPT // LANES


def _prep_body(ids_ref, msk_ref, sel_ref, inv_ref):
    ids = ids_ref[...]
    msk = msk_ref[...]
    sel = jnp.where(msk > 0, ids, DEAD)
    selp = jnp.concatenate(
        [sel, jnp.full((SP - S, PBLK), DEAD, jnp.int32)], axis=0)
    sel_ref[...] = selp.T
    cnt = jnp.sum(msk.astype(jnp.float32), axis=0)
    inv = 1.0 / jnp.maximum(cnt, 1.0)
    inv_ref[...] = jnp.broadcast_to(inv[None, :], (8, PBLK))


_prep = pl.pallas_call(
    _prep_body,
    grid=(B // PBLK,),
    in_specs=[
        pl.BlockSpec((S, PBLK), lambda i: (0, i)),
        pl.BlockSpec((S, PBLK), lambda i: (0, i)),
    ],
    out_specs=[
        pl.BlockSpec((PBLK, SP), lambda i: (i, 0)),
        pl.BlockSpec((8, PBLK), lambda i: (0, i)),
    ],
    out_shape=[
        jax.ShapeDtypeStruct((B, SP), jnp.int32),
        jax.ShapeDtypeStruct((8, B), jnp.float32),
    ],
)


def _proj_body(w_ref, embt_ref, out_ref):
    i = pl.program_id(0)
    y = lax.dot_general(w_ref[...], embt_ref[...],
                        (((1,), (0,)), ((), ())),
                        preferred_element_type=jnp.float32)
    cols = i * CBLK + lax.broadcasted_iota(jnp.int32, (L, CBLK), 1)
    y = jnp.where(cols < V, y, 0.0)
    yb = y.astype(jnp.bfloat16)
    lo = lax.bitcast_convert_type(yb[0, :], jnp.uint16).astype(jnp.uint32)
    hi = lax.bitcast_convert_type(yb[1, :], jnp.uint16).astype(jnp.uint32)
    out_ref[...] = lax.bitcast_convert_type(lo | (hi << 16), jnp.int32)


_proj_table = pl.pallas_call(
    _proj_body,
    grid=(VP // CBLK,),
    in_specs=[
        pl.BlockSpec((L, D), lambda i: (0, 0)),
        pl.BlockSpec((D, CBLK), lambda i: (0, i)),
    ],
    out_specs=pl.BlockSpec((CBLK,), lambda i: (i,)),
    out_shape=jax.ShapeDtypeStruct((VP,), jnp.int32),
)


def _rot(x, idx):
    return lax.gather(
        x, idx[:, None],
        lax.GatherDimensionNumbers(
            offset_dims=(), collapsed_slice_dims=(0,), start_index_map=(0,)),
        (1,), mode=lax.GatherScatterMode.PROMISE_IN_BOUNDS)


def _sc_body(tbl_hbm, sel_hbm, inv_hbm, bias_hbm, out_hbm,
             tbl_v, sel_v, inv_v, bias_v, out_v, sems):
    wid = lax.axis_index("s") * NC + lax.axis_index("c")
    row0 = wid * RPT

    pltpu.sync_copy(tbl_hbm, tbl_v)
    pltpu.sync_copy(inv_hbm.at[0, pl.ds(row0, RPT)], inv_v)
    pltpu.sync_copy(bias_hbm, bias_v)

    def issue(slab, buf):
        pltpu.make_async_copy(
            sel_hbm.at[pl.ds(row0 + slab * LANES, LANES)],
            sel_v.at[buf], sems[buf]).start()

    def drain(buf):
        pltpu.make_async_copy(
            sel_hbm.at[pl.ds(row0, LANES)],
            sel_v.at[buf], sems[buf]).wait()

    issue(0, 0)
    issue(1, 1)

    lane = lax.iota(jnp.int32, LANES)
    rot8 = (lane + 8) & 15
    rot4 = (lane + 4) & 15
    rot2 = (lane + 2) & 15
    rot1 = (lane + 1) & 15
    zero = jnp.zeros((LANES,), jnp.float32)
    bias0 = _rot(bias_v[...], jnp.zeros((LANES,), jnp.int32))
    bias1 = _rot(bias_v[...], jnp.ones((LANES,), jnp.int32))

    def tree(x):
        x = x + _rot(x, rot8)
        x = x + _rot(x, rot4)
        x = x + _rot(x, rot2)
        return x + _rot(x, rot1)

    def slab_group(g, carry):
        for buf in range(2):
            slab = g * 2 + buf
            drain(buf)

            def row_body(r, regs):
                sr0, sr1 = regs
                a = [zero, zero, zero, zero]
                b = [zero, zero, zero, zero]
                for c in range(SP // LANES):
                    sel16 = sel_v[buf, r, pl.ds(c * LANES, LANES)]
                    g16 = plsc.load_gather(tbl_v, [sel16])
                    pair = plsc.bitcast(g16, jnp.bfloat16)
                    u0, u1 = plsc.unpack(pair,
                                         format=plsc.PackFormat.INTERLEAVED)
                    a[c % 4] = a[c % 4] + u0
                    b[c % 4] = b[c % 4] + u1
                t0 = tree((a[0] + a[1]) + (a[2] + a[3]))
                t1 = tree((b[0] + b[1]) + (b[2] + b[3]))
                return (jnp.where(lane == r, t0, sr0),
                        jnp.where(lane == r, t1, sr1))
            sr0, sr1 = lax.fori_loop(0, LANES, row_body, (zero, zero))

            @pl.when(slab + 2 < NSLAB)
            def _():
                issue(slab + 2, buf)

            sl = pl.ds(slab * LANES, LANES)
            iv = inv_v[sl]
            out_v[0, sl] = sr0 * iv + bias0
            out_v[1, sl] = sr1 * iv + bias1
        return carry
    lax.fori_loop(0, NSLAB // 2, slab_group, 0)

    pltpu.sync_copy(out_v.at[0], out_hbm.at[0, pl.ds(row0, RPT)])
    pltpu.sync_copy(out_v.at[1], out_hbm.at[1, pl.ds(row0, RPT)])


_sc_logits = functools.partial(
    pl.kernel,
    out_type=jax.ShapeDtypeStruct((L, B), jnp.float32),
    mesh=plsc.VectorSubcoreMesh(core_axis_name="c", subcore_axis_name="s"),
    compiler_params=pltpu.CompilerParams(
        use_tc_tiling_on_sc=False, needs_layout_passes=False),
    scratch_types=[
        pltpu.VMEM((VP,), jnp.int32),
        pltpu.VMEM((2, LANES, SP), jnp.int32),
        pltpu.VMEM((RPT,), jnp.float32),
        pltpu.VMEM((LANES,), jnp.float32),
        pltpu.VMEM((L, RPT), jnp.float32),
        [pltpu.SemaphoreType.DMA] * 2,
    ],
)(_sc_body)


def kernel(input_ids, attention_mask, emb_weight, fc_w, fc_b):
    sel, inv8 = _prep(input_ids.T.astype(jnp.int32),
                      attention_mask.T.astype(jnp.int32))
    tbl = _proj_table(fc_w.astype(jnp.float32), emb_weight.T)
    bias16 = jnp.zeros((LANES,), jnp.float32).at[:L].set(fc_b)
    out2 = _sc_logits(tbl, sel, inv8, bias16)
    return out2.T


# speedup vs baseline: 51.9256x; 1.0965x over previous
import functools

import jax
import jax.numpy as jnp
from jax import lax
from jax.experimental import pallas as pl
from jax.experimental.pallas import tpu as pltpu
from jax.experimental.pallas import tpu_sc as plsc

V, D, L = 100000, 64, 2
B, S = 4096, 200

NC, NS, LANES = 2, 16, 16
NW = NC * NS
DEAD = V
VP = 102400
CBLK = 51200
SP = 208
RPT = B // NW
PBLK = 512
NSLAB = RPT // LANES


def _prep_body(ids_ref, msk_ref, sela_ref, selb_ref, inv_ref):
    ids = ids_ref[...]
    msk = msk_ref[...]
    sel = jnp.where(msk > 0, ids, DEAD)
    selp = jnp.concatenate(
        [sel, jnp.full((256 - S, PBLK), DEAD, jnp.int32)], axis=0)
    sela_ref[...] = selp[:128].T
    selb_ref[...] = selp[128:].T
    cnt = jnp.sum(msk.astype(jnp.float32), axis=0)
    inv = 1.0 / jnp.maximum(cnt, 1.0)
    inv_ref[...] = jnp.broadcast_to(inv[None, :], (8, PBLK))


_prep = pl.pallas_call(
    _prep_body,
    grid=(B // PBLK,),
    in_specs=[
        pl.BlockSpec((S, PBLK), lambda i: (0, i)),
        pl.BlockSpec((S, PBLK), lambda i: (0, i)),
    ],
    out_specs=[
        pl.BlockSpec((PBLK, 128), lambda i: (i, 0)),
        pl.BlockSpec((PBLK, 128), lambda i: (i, 0)),
        pl.BlockSpec((8, PBLK), lambda i: (0, i)),
    ],
    out_shape=[
        jax.ShapeDtypeStruct((B, 128), jnp.int32),
        jax.ShapeDtypeStruct((B, 128), jnp.int32),
        jax.ShapeDtypeStruct((8, B), jnp.float32),
    ],
)


def _proj_body(w_ref, embt_ref, out_ref):
    i = pl.program_id(0)
    y = lax.dot_general(w_ref[...], embt_ref[...],
                        (((1,), (0,)), ((), ())),
                        preferred_element_type=jnp.float32)
    cols = i * CBLK + lax.broadcasted_iota(jnp.int32, (L, CBLK), 1)
    y = jnp.where(cols < V, y, 0.0)
    yb = y.astype(jnp.bfloat16)
    lo = lax.bitcast_convert_type(yb[0, :], jnp.uint16).astype(jnp.uint32)
    hi = lax.bitcast_convert_type(yb[1, :], jnp.uint16).astype(jnp.uint32)
    out_ref[...] = lax.bitcast_convert_type(lo | (hi << 16), jnp.int32)


_proj_table = pl.pallas_call(
    _proj_body,
    grid=(VP // CBLK,),
    in_specs=[
        pl.BlockSpec((L, D), lambda i: (0, 0)),
        pl.BlockSpec((D, CBLK), lambda i: (0, i)),
    ],
    out_specs=pl.BlockSpec((CBLK,), lambda i: (i,)),
    out_shape=jax.ShapeDtypeStruct((VP,), jnp.int32),
)


def _rot(x, idx):
    return lax.gather(
        x, idx[:, None],
        lax.GatherDimensionNumbers(
            offset_dims=(), collapsed_slice_dims=(0,), start_index_map=(0,)),
        (1,), mode=lax.GatherScatterMode.PROMISE_IN_BOUNDS)


def _sc_body(tbl_hbm, sela_hbm, selb_hbm, inv_hbm, bias_hbm, out_hbm,
             tbl_v, sel_v, inv_v, bias_v, out_v, sems):
    wid = lax.axis_index("s") * NC + lax.axis_index("c")
    row0 = wid * RPT

    pltpu.sync_copy(tbl_hbm, tbl_v)
    pltpu.sync_copy(inv_hbm.at[0, pl.ds(row0, RPT)], inv_v)
    pltpu.sync_copy(bias_hbm, bias_v)

    def issue(slab, buf):
        rows = pl.ds(row0 + slab * LANES, LANES)
        pltpu.make_async_copy(
            sela_hbm.at[rows], sel_v.at[buf, 0], sems[buf]).start()
        pltpu.make_async_copy(
            selb_hbm.at[rows], sel_v.at[buf, 1], sems[buf]).start()

    def drain(buf):
        rows = pl.ds(row0, LANES)
        pltpu.make_async_copy(
            sela_hbm.at[rows], sel_v.at[buf, 0], sems[buf]).wait()
        pltpu.make_async_copy(
            selb_hbm.at[rows], sel_v.at[buf, 1], sems[buf]).wait()

    issue(0, 0)
    issue(1, 1)

    lane = lax.iota(jnp.int32, LANES)
    rot8 = (lane + 8) & 15
    rot4 = (lane + 4) & 15
    rot2 = (lane + 2) & 15
    rot1 = (lane + 1) & 15
    zero = jnp.zeros((LANES,), jnp.float32)
    bias0 = _rot(bias_v[...], jnp.zeros((LANES,), jnp.int32))
    bias1 = _rot(bias_v[...], jnp.ones((LANES,), jnp.int32))

    def tree(x):
        x = x + _rot(x, rot8)
        x = x + _rot(x, rot4)
        x = x + _rot(x, rot2)
        return x + _rot(x, rot1)

    def slab_group(g, carry):
        for buf in range(2):
            slab = g * 2 + buf
            drain(buf)

            def row_body(r, regs):
                sr0, sr1 = regs
                a = [zero, zero, zero, zero]
                b = [zero, zero, zero, zero]
                for c in range(SP // LANES):
                    half, cc = (0, c) if c < 8 else (1, c - 8)
                    sel16 = sel_v[buf, half, r, pl.ds(cc * LANES, LANES)]
                    g16 = plsc.load_gather(tbl_v, [sel16])
                    pair = plsc.bitcast(g16, jnp.bfloat16)
                    u0, u1 = plsc.unpack(pair,
                                         format=plsc.PackFormat.INTERLEAVED)
                    a[c % 4] = a[c % 4] + u0
                    b[c % 4] = b[c % 4] + u1
                t0 = tree((a[0] + a[1]) + (a[2] + a[3]))
                t1 = tree((b[0] + b[1]) + (b[2] + b[3]))
                return (jnp.where(lane == r, t0, sr0),
                        jnp.where(lane == r, t1, sr1))
            sr0, sr1 = lax.fori_loop(0, LANES, row_body, (zero, zero))

            @pl.when(slab + 2 < NSLAB)
            def _():
                issue(slab + 2, buf)

            sl = pl.ds(slab * LANES, LANES)
            iv = inv_v[sl]
            out_v[0, sl] = sr0 * iv + bias0
            out_v[1, sl] = sr1 * iv + bias1
        return carry
    lax.fori_loop(0, NSLAB // 2, slab_group, 0)

    pltpu.sync_copy(out_v.at[0], out_hbm.at[0, pl.ds(row0, RPT)])
    pltpu.sync_copy(out_v.at[1], out_hbm.at[1, pl.ds(row0, RPT)])


_sc_logits = functools.partial(
    pl.kernel,
    out_type=jax.ShapeDtypeStruct((L, B), jnp.float32),
    mesh=plsc.VectorSubcoreMesh(core_axis_name="c", subcore_axis_name="s"),
    compiler_params=pltpu.CompilerParams(
        use_tc_tiling_on_sc=False, needs_layout_passes=False),
    scratch_types=[
        pltpu.VMEM((VP,), jnp.int32),
        pltpu.VMEM((2, 2, LANES, 128), jnp.int32),
        pltpu.VMEM((RPT,), jnp.float32),
        pltpu.VMEM((LANES,), jnp.float32),
        pltpu.VMEM((L, RPT), jnp.float32),
        [pltpu.SemaphoreType.DMA] * 2,
    ],
)(_sc_body)


def kernel(input_ids, attention_mask, emb_weight, fc_w, fc_b):
    sela, selb, inv8 = _prep(input_ids.T.astype(jnp.int32),
                             attention_mask.T.astype(jnp.int32))
    tbl = _proj_table(fc_w.astype(jnp.float32), emb_weight.T)
    bias16 = jnp.zeros((LANES,), jnp.float32).at[:L].set(fc_b)
    out2 = _sc_logits(tbl, sela, selb, inv8, bias16)
    return out2.T


# speedup vs baseline: 52.1797x; 1.0049x over previous
import functools

import jax
import jax.numpy as jnp
from jax import lax
from jax.experimental import pallas as pl
from jax.experimental.pallas import tpu as pltpu
from jax.experimental.pallas import tpu_sc as plsc

V, D, L = 100000, 64, 2
B, S = 4096, 200

NC, NS, LANES = 2, 16, 16
NW = NC * NS
DEAD = V
VP = 102400
CBLK = 51200
SP = 208
RPT = B // NW
PBLK = 512
NSLAB = RPT // LANES


def _prep_body(ids_ref, msk_ref, sela_ref, selb_ref, inv_ref):
    ids = ids_ref[...]
    msk = msk_ref[...]
    sel = jnp.where(msk > 0, ids, DEAD)
    selp = jnp.concatenate(
        [sel, jnp.full((256 - S, PBLK), DEAD, jnp.int32)], axis=0)
    sela_ref[...] = selp[:128].T
    selb_ref[...] = selp[128:].T
    cnt = jnp.sum(msk.astype(jnp.float32), axis=0)
    inv = 1.0 / jnp.maximum(cnt, 1.0)
    inv_ref[...] = jnp.broadcast_to(inv[None, :], (8, PBLK))


_prep = pl.pallas_call(
    _prep_body,
    grid=(B // PBLK,),
    in_specs=[
        pl.BlockSpec((S, PBLK), lambda i: (0, i)),
        pl.BlockSpec((S, PBLK), lambda i: (0, i)),
    ],
    out_specs=[
        pl.BlockSpec((PBLK, 128), lambda i: (i, 0)),
        pl.BlockSpec((PBLK, 128), lambda i: (i, 0)),
        pl.BlockSpec((8, PBLK), lambda i: (0, i)),
    ],
    out_shape=[
        jax.ShapeDtypeStruct((B, 128), jnp.int32),
        jax.ShapeDtypeStruct((B, 128), jnp.int32),
        jax.ShapeDtypeStruct((8, B), jnp.float32),
    ],
)


def _proj_body(w_ref, embt_ref, out_ref):
    i = pl.program_id(0)
    y = lax.dot_general(w_ref[...], embt_ref[...],
                        (((1,), (0,)), ((), ())),
                        preferred_element_type=jnp.float32)
    cols = i * CBLK + lax.broadcasted_iota(jnp.int32, (L, CBLK), 1)
    y = jnp.where(cols < V, y, 0.0)
    yb = y.astype(jnp.bfloat16)
    lo = lax.bitcast_convert_type(yb[0, :], jnp.uint16).astype(jnp.uint32)
    hi = lax.bitcast_convert_type(yb[1, :], jnp.uint16).astype(jnp.uint32)
    out_ref[...] = lax.bitcast_convert_type(lo | (hi << 16), jnp.int32)


_proj_table = pl.pallas_call(
    _proj_body,
    grid=(VP // CBLK,),
    in_specs=[
        pl.BlockSpec((L, D), lambda i: (0, 0)),
        pl.BlockSpec((D, CBLK), lambda i: (0, i)),
    ],
    out_specs=pl.BlockSpec((CBLK,), lambda i: (i,)),
    out_shape=jax.ShapeDtypeStruct((VP,), jnp.int32),
)


def _rot(x, idx):
    return lax.gather(
        x, idx[:, None],
        lax.GatherDimensionNumbers(
            offset_dims=(), collapsed_slice_dims=(0,), start_index_map=(0,)),
        (1,), mode=lax.GatherScatterMode.PROMISE_IN_BOUNDS)


def _sc_body(tbl_hbm, sela_hbm, selb_hbm, inv_hbm, bias_hbm, out_hbm,
             tbl_v, sel_v, inv_v, bias_v, out_v, sems):
    wid = lax.axis_index("s") * NC + lax.axis_index("c")
    row0 = wid * RPT

    pltpu.sync_copy(tbl_hbm, tbl_v)
    pltpu.sync_copy(inv_hbm.at[0, pl.ds(row0, RPT)], inv_v)
    pltpu.sync_copy(bias_hbm, bias_v)

    def issue(slab, buf):
        rows = pl.ds(row0 + slab * LANES, LANES)
        pltpu.make_async_copy(
            sela_hbm.at[rows], sel_v.at[buf, 0], sems[buf]).start()
        pltpu.make_async_copy(
            selb_hbm.at[rows], sel_v.at[buf, 1], sems[buf]).start()

    def drain(buf):
        rows = pl.ds(row0, LANES)
        pltpu.make_async_copy(
            sela_hbm.at[rows], sel_v.at[buf, 0], sems[buf]).wait()
        pltpu.make_async_copy(
            selb_hbm.at[rows], sel_v.at[buf, 1], sems[buf]).wait()

    issue(0, 0)
    issue(1, 1)

    lane = lax.iota(jnp.int32, LANES)
    rot8 = (lane + 8) & 15
    rot4 = (lane + 4) & 15
    rot2 = (lane + 2) & 15
    rot1 = (lane + 1) & 15
    zero = jnp.zeros((LANES,), jnp.float32)
    bias0 = _rot(bias_v[...], jnp.zeros((LANES,), jnp.int32))
    bias1 = _rot(bias_v[...], jnp.ones((LANES,), jnp.int32))

    def tree(x):
        x = x + _rot(x, rot8)
        x = x + _rot(x, rot4)
        x = x + _rot(x, rot2)
        return x + _rot(x, rot1)

    def slab_group(g, carry):
        for buf in range(2):
            slab = g * 2 + buf
            drain(buf)

            def row_body(r2, regs):
                sr0, sr1 = regs
                for k in range(2):
                    r = r2 * 2 + k
                    a = [zero, zero, zero, zero]
                    b = [zero, zero, zero, zero]
                    for c in range(SP // LANES):
                        half, cc = (0, c) if c < 8 else (1, c - 8)
                        sel16 = sel_v[buf, half, r, pl.ds(cc * LANES, LANES)]
                        g16 = plsc.load_gather(tbl_v, [sel16])
                        pair = plsc.bitcast(g16, jnp.bfloat16)
                        u0, u1 = plsc.unpack(
                            pair, format=plsc.PackFormat.INTERLEAVED)
                        a[c % 4] = a[c % 4] + u0
                        b[c % 4] = b[c % 4] + u1
                    t0 = tree((a[0] + a[1]) + (a[2] + a[3]))
                    t1 = tree((b[0] + b[1]) + (b[2] + b[3]))
                    sr0 = jnp.where(lane == r, t0, sr0)
                    sr1 = jnp.where(lane == r, t1, sr1)
                return (sr0, sr1)
            sr0, sr1 = lax.fori_loop(0, LANES // 2, row_body, (zero, zero))

            @pl.when(slab + 2 < NSLAB)
            def _():
                issue(slab + 2, buf)

            sl = pl.ds(slab * LANES, LANES)
            iv = inv_v[sl]
            out_v[0, sl] = sr0 * iv + bias0
            out_v[1, sl] = sr1 * iv + bias1
        return carry
    lax.fori_loop(0, NSLAB // 2, slab_group, 0)

    pltpu.sync_copy(out_v.at[0], out_hbm.at[0, pl.ds(row0, RPT)])
    pltpu.sync_copy(out_v.at[1], out_hbm.at[1, pl.ds(row0, RPT)])


_sc_logits = functools.partial(
    pl.kernel,
    out_type=jax.ShapeDtypeStruct((L, B), jnp.float32),
    mesh=plsc.VectorSubcoreMesh(core_axis_name="c", subcore_axis_name="s"),
    compiler_params=pltpu.CompilerParams(
        use_tc_tiling_on_sc=False, needs_layout_passes=False),
    scratch_types=[
        pltpu.VMEM((VP,), jnp.int32),
        pltpu.VMEM((2, 2, LANES, 128), jnp.int32),
        pltpu.VMEM((RPT,), jnp.float32),
        pltpu.VMEM((LANES,), jnp.float32),
        pltpu.VMEM((L, RPT), jnp.float32),
        [pltpu.SemaphoreType.DMA] * 2,
    ],
)(_sc_body)


def kernel(input_ids, attention_mask, emb_weight, fc_w, fc_b):
    sela, selb, inv8 = _prep(input_ids.T.astype(jnp.int32),
                             attention_mask.T.astype(jnp.int32))
    tbl = _proj_table(fc_w.astype(jnp.float32), emb_weight.T)
    bias16 = jnp.zeros((LANES,), jnp.float32).at[:L].set(fc_b)
    out2 = _sc_logits(tbl, sela, selb, inv8, bias16)
    return out2.T
